# baseline (speedup 1.0000x reference)
import functools
import jax
import jax.numpy as jnp
import numpy as np
from jax import lax
from jax.experimental import pallas as pl
from jax.experimental.pallas import tpu as pltpu

B, N, C = 4, 16384, 64
NPOINT = 1024
RADII = [0.5, 1.0, 2.0]
NSAMPLES = [16, 16, 32]
EPS = 1e-5

_CT = 128


def _d2_body(px_ref, py_ref, pz_ref, cx_ref, cy_ref, cz_ref, o_ref):
    dx = cx_ref[...] - px_ref[...]
    dy = cy_ref[...] - py_ref[...]
    dz = cz_ref[...] - pz_ref[...]
    o_ref[...] = dx * dx + dy * dy + dz * dz


def _pairwise_d2(xyz, new_xyz):
    px = xyz[:, :, 0][:, None, :]
    py = xyz[:, :, 1][:, None, :]
    pz = xyz[:, :, 2][:, None, :]
    cx = new_xyz[:, :, 0][:, :, None]
    cy = new_xyz[:, :, 1][:, :, None]
    cz = new_xyz[:, :, 2][:, :, None]
    pt_spec = pl.BlockSpec((1, 1, N), lambda b, m: (b, 0, 0))
    c_spec = pl.BlockSpec((1, _CT, 1), lambda b, m: (b, m, 0))
    return pl.pallas_call(
        _d2_body,
        grid=(B, NPOINT // _CT),
        in_specs=[pt_spec, pt_spec, pt_spec, c_spec, c_spec, c_spec],
        out_specs=pl.BlockSpec((1, _CT, N), lambda b, m: (b, m, 0)),
        out_shape=jax.ShapeDtypeStruct((B, NPOINT, N), jnp.float32),
    )(px, py, pz, cx, cy, cz)


def _fps(xyz, npoint):
    b, n, _ = xyz.shape

    def body(i, state):
        dists, farthest, idxs = state
        idxs = idxs.at[:, i].set(farthest)
        centroid = jnp.take_along_axis(xyz, farthest[:, None, None], axis=1)
        d = jnp.sum((xyz - centroid) ** 2, axis=-1)
        dists = jnp.minimum(dists, d)
        farthest = jnp.argmax(dists, axis=-1).astype(jnp.int32)
        return (dists, farthest, idxs)

    state = (jnp.full((b, n), 1e10, jnp.float32), jnp.zeros((b,), jnp.int32),
             jnp.zeros((b, npoint), jnp.int32))
    state = jax.lax.fori_loop(0, npoint, body, state)
    return state[2]


def _group(idx, xyz, new_xyz, features):
    grouped_xyz = jax.vmap(lambda pts, ix: pts[ix])(xyz, idx)
    grouped_xyz = grouped_xyz - new_xyz[:, :, None, :]
    grouped_xyz = jnp.transpose(grouped_xyz, (0, 3, 1, 2))
    grouped_feat = jax.vmap(lambda f, ix: f[:, ix])(features, idx)
    return jnp.concatenate([grouped_xyz, grouped_feat], axis=1)


def _mlp_branch(x, layers):
    for L in layers:
        x = jnp.einsum('oc,bcms->boms', L["W"], x)
        mean = jnp.mean(x, axis=(0, 2, 3), keepdims=True)
        var = jnp.var(x, axis=(0, 2, 3), keepdims=True)
        x = (x - mean) / jnp.sqrt(var + EPS)
        x = x * L["g"][None, :, None, None] + L["b"][None, :, None, None]
        x = jax.nn.relu(x)
    return x


def _out_kernel(x_ref, w_ref, g_ref, b_ref, o_ref):
    ys = []
    for b in range(B):
        ys.append(jnp.dot(w_ref[...], x_ref[b], preferred_element_type=jnp.float32))
    y = jnp.stack(ys, axis=0)
    cnt = y.shape[0] * y.shape[2]
    mean = jnp.sum(y, axis=(0, 2)) / cnt
    var = jnp.sum(y * y, axis=(0, 2)) / cnt - mean * mean
    scale = g_ref[...] / jnp.sqrt(var + EPS)
    shift = b_ref[...] - mean * scale
    o_ref[...] = jnp.maximum(y * scale[None, :, None] + shift[None, :, None], 0.0)


def _out_layer(nf, L):
    O, Cin = L["W"].shape
    M = nf.shape[2]
    return pl.pallas_call(
        _out_kernel,
        out_shape=jax.ShapeDtypeStruct((B, O, M), jnp.float32),
    )(nf, L["W"], L["g"], L["b"])


def kernel(xyz, features, params):
    fps_idx = _fps(xyz, NPOINT)
    new_xyz = jax.vmap(lambda pts, ix: pts[ix])(xyz, fps_idx)
    d2 = _pairwise_d2(xyz, new_xyz)
    arange_n = jnp.arange(N, dtype=jnp.int32)[None, None, :]
    feats = []
    for i in range(len(RADII)):
        key = jnp.where(d2 <= RADII[i] * RADII[i], arange_n, N)
        idx_sorted = jnp.sort(key, axis=-1)[..., :NSAMPLES[i]]
        first = idx_sorted[..., :1]
        idx = jnp.where(idx_sorted >= N, jnp.broadcast_to(first, idx_sorted.shape), idx_sorted)
        idx = jnp.where(idx >= N, 0, idx).astype(jnp.int32)
        g = _group(idx, xyz, new_xyz, features)
        h = _mlp_branch(g, params["branches"][i])
        h = jnp.max(h, axis=-1)
        feats.append(h)
    nf = jnp.concatenate(feats, axis=1)
    nf = _out_layer(nf, params["out"])
    return (new_xyz, nf)


# speedup vs baseline: 1.2452x; 1.2452x over previous
import functools
import jax
import jax.numpy as jnp
import numpy as np
from jax import lax
from jax.experimental import pallas as pl
from jax.experimental.pallas import tpu as pltpu

B, N, C = 4, 16384, 64
NPOINT = 1024
RADII = [0.5, 1.0, 2.0]
NSAMPLES = [16, 16, 32]
EPS = 1e-5

_CT = 128


def _d2_body(px_ref, py_ref, pz_ref, cx_ref, cy_ref, cz_ref, o_ref):
    dx = cx_ref[...] - px_ref[...]
    dy = cy_ref[...] - py_ref[...]
    dz = cz_ref[...] - pz_ref[...]
    o_ref[...] = dx * dx + dy * dy + dz * dz


def _pairwise_d2(xyz, new_xyz):
    px = xyz[:, :, 0][:, None, :]
    py = xyz[:, :, 1][:, None, :]
    pz = xyz[:, :, 2][:, None, :]
    cx = new_xyz[:, :, 0][:, :, None]
    cy = new_xyz[:, :, 1][:, :, None]
    cz = new_xyz[:, :, 2][:, :, None]
    pt_spec = pl.BlockSpec((1, 1, N), lambda b, m: (b, 0, 0))
    c_spec = pl.BlockSpec((1, _CT, 1), lambda b, m: (b, m, 0))
    return pl.pallas_call(
        _d2_body,
        grid=(B, NPOINT // _CT),
        in_specs=[pt_spec, pt_spec, pt_spec, c_spec, c_spec, c_spec],
        out_specs=pl.BlockSpec((1, _CT, N), lambda b, m: (b, m, 0)),
        out_shape=jax.ShapeDtypeStruct((B, NPOINT, N), jnp.float32),
    )(px, py, pz, cx, cy, cz)


_FR, _FC = 8, N // 8


def _fps_body(px_ref, py_ref, pz_ref, o_ref, dist_ref):
    nmat = (lax.broadcasted_iota(jnp.int32, (1, _FR, _FC), 1) * _FC
            + lax.broadcasted_iota(jnp.int32, (1, _FR, _FC), 2))
    px = px_ref[...]
    py = py_ref[...]
    pz = pz_ref[...]
    dist_ref[...] = jnp.full((1, _FR, _FC), 1e10, jnp.float32)

    iota_np = lax.broadcasted_iota(jnp.int32, (1, NPOINT), 1)

    def body(i, st):
        farthest, acc = st
        acc = jnp.where(iota_np == i, farthest, acc)
        sel = nmat == farthest
        zf = jnp.zeros((1, _FR, _FC), jnp.float32)
        cx = jnp.sum(jnp.where(sel, px, zf))
        cy = jnp.sum(jnp.where(sel, py, zf))
        cz = jnp.sum(jnp.where(sel, pz, zf))
        dx = px - cx
        dy = py - cy
        dz = pz - cz
        d = jnp.minimum(dist_ref[...], dx * dx + dy * dy + dz * dz)
        dist_ref[...] = d
        m = jnp.max(d)
        nxt = jnp.min(jnp.where(d == m, nmat, N)).astype(jnp.int32)
        return (nxt, acc)

    init = (jnp.int32(0), jnp.zeros((1, NPOINT), jnp.int32))
    _, acc = lax.fori_loop(0, NPOINT, body, init)
    o_ref[0] = acc


def _fps(xyz, npoint):
    px = xyz[:, :, 0].reshape(B, _FR, _FC)
    py = xyz[:, :, 1].reshape(B, _FR, _FC)
    pz = xyz[:, :, 2].reshape(B, _FR, _FC)
    spec = pl.BlockSpec((1, _FR, _FC), lambda b: (b, 0, 0))
    out = pl.pallas_call(
        _fps_body,
        grid=(B,),
        in_specs=[spec, spec, spec],
        out_specs=pl.BlockSpec((1, 1, npoint), lambda b: (b, 0, 0)),
        out_shape=jax.ShapeDtypeStruct((B, 1, npoint), jnp.int32),
        scratch_shapes=[pltpu.VMEM((1, _FR, _FC), jnp.float32)],
    )(px, py, pz)
    return out[:, 0, :]


def _group(idx, xyz, new_xyz, features):
    grouped_xyz = jax.vmap(lambda pts, ix: pts[ix])(xyz, idx)
    grouped_xyz = grouped_xyz - new_xyz[:, :, None, :]
    grouped_xyz = jnp.transpose(grouped_xyz, (0, 3, 1, 2))
    grouped_feat = jax.vmap(lambda f, ix: f[:, ix])(features, idx)
    return jnp.concatenate([grouped_xyz, grouped_feat], axis=1)


def _mlp_branch(x, layers):
    for L in layers:
        x = jnp.einsum('oc,bcms->boms', L["W"], x)
        mean = jnp.mean(x, axis=(0, 2, 3), keepdims=True)
        var = jnp.var(x, axis=(0, 2, 3), keepdims=True)
        x = (x - mean) / jnp.sqrt(var + EPS)
        x = x * L["g"][None, :, None, None] + L["b"][None, :, None, None]
        x = jax.nn.relu(x)
    return x


def _out_kernel(x_ref, w_ref, g_ref, b_ref, o_ref):
    ys = []
    for b in range(B):
        ys.append(jnp.dot(w_ref[...], x_ref[b], preferred_element_type=jnp.float32))
    y = jnp.stack(ys, axis=0)
    cnt = y.shape[0] * y.shape[2]
    mean = jnp.sum(y, axis=(0, 2)) / cnt
    var = jnp.sum(y * y, axis=(0, 2)) / cnt - mean * mean
    scale = g_ref[...] / jnp.sqrt(var + EPS)
    shift = b_ref[...] - mean * scale
    o_ref[...] = jnp.maximum(y * scale[None, :, None] + shift[None, :, None], 0.0)


def _out_layer(nf, L):
    O, Cin = L["W"].shape
    M = nf.shape[2]
    return pl.pallas_call(
        _out_kernel,
        out_shape=jax.ShapeDtypeStruct((B, O, M), jnp.float32),
    )(nf, L["W"], L["g"], L["b"])


def kernel(xyz, features, params):
    fps_idx = _fps(xyz, NPOINT)
    new_xyz = jax.vmap(lambda pts, ix: pts[ix])(xyz, fps_idx)
    d2 = _pairwise_d2(xyz, new_xyz)
    arange_n = jnp.arange(N, dtype=jnp.int32)[None, None, :]
    feats = []
    for i in range(len(RADII)):
        key = jnp.where(d2 <= RADII[i] * RADII[i], arange_n, N)
        negv, _ = lax.top_k(-key, NSAMPLES[i])
        idx_sorted = -negv
        first = idx_sorted[..., :1]
        idx = jnp.where(idx_sorted >= N, jnp.broadcast_to(first, idx_sorted.shape), idx_sorted)
        idx = jnp.where(idx >= N, 0, idx).astype(jnp.int32)
        g = _group(idx, xyz, new_xyz, features)
        h = _mlp_branch(g, params["branches"][i])
        h = jnp.max(h, axis=-1)
        feats.append(h)
    nf = jnp.concatenate(feats, axis=1)
    nf = _out_layer(nf, params["out"])
    return (new_xyz, nf)


# speedup vs baseline: 4.9882x; 4.0060x over previous
import functools
import jax
import jax.numpy as jnp
import numpy as np
from jax import lax
from jax.experimental import pallas as pl
from jax.experimental.pallas import tpu as pltpu

B, N, C = 4, 16384, 64
NPOINT = 1024
RADII = [0.5, 1.0, 2.0]
NSAMPLES = [16, 16, 32]
EPS = 1e-5

_CT = 128


def _d2_body(px_ref, py_ref, pz_ref, cx_ref, cy_ref, cz_ref, o_ref):
    dx = cx_ref[...] - px_ref[...]
    dy = cy_ref[...] - py_ref[...]
    dz = cz_ref[...] - pz_ref[...]
    o_ref[...] = dx * dx + dy * dy + dz * dz


def _pairwise_d2(xyz, new_xyz):
    px = xyz[:, :, 0][:, None, :]
    py = xyz[:, :, 1][:, None, :]
    pz = xyz[:, :, 2][:, None, :]
    cx = new_xyz[:, :, 0][:, :, None]
    cy = new_xyz[:, :, 1][:, :, None]
    cz = new_xyz[:, :, 2][:, :, None]
    pt_spec = pl.BlockSpec((1, 1, N), lambda b, m: (b, 0, 0))
    c_spec = pl.BlockSpec((1, _CT, 1), lambda b, m: (b, m, 0))
    return pl.pallas_call(
        _d2_body,
        grid=(B, NPOINT // _CT),
        in_specs=[pt_spec, pt_spec, pt_spec, c_spec, c_spec, c_spec],
        out_specs=pl.BlockSpec((1, _CT, N), lambda b, m: (b, m, 0)),
        out_shape=jax.ShapeDtypeStruct((B, NPOINT, N), jnp.float32),
    )(px, py, pz, cx, cy, cz)


_FR, _FC = 8, N // 8


def _fps_body(px_ref, py_ref, pz_ref, o_ref, dist_ref):
    nmat = (lax.broadcasted_iota(jnp.int32, (1, _FR, _FC), 1) * _FC
            + lax.broadcasted_iota(jnp.int32, (1, _FR, _FC), 2))
    px = px_ref[...]
    py = py_ref[...]
    pz = pz_ref[...]
    dist_ref[...] = jnp.full((1, _FR, _FC), 1e10, jnp.float32)

    iota_np = lax.broadcasted_iota(jnp.int32, (1, NPOINT), 1)

    def body(i, st):
        farthest, acc = st
        acc = jnp.where(iota_np == i, farthest, acc)
        sel = nmat == farthest
        zf = jnp.zeros((1, _FR, _FC), jnp.float32)
        cx = jnp.sum(jnp.where(sel, px, zf))
        cy = jnp.sum(jnp.where(sel, py, zf))
        cz = jnp.sum(jnp.where(sel, pz, zf))
        dx = px - cx
        dy = py - cy
        dz = pz - cz
        d = jnp.minimum(dist_ref[...], dx * dx + dy * dy + dz * dz)
        dist_ref[...] = d
        m = jnp.max(d)
        nxt = jnp.min(jnp.where(d == m, nmat, N)).astype(jnp.int32)
        return (nxt, acc)

    init = (jnp.int32(0), jnp.zeros((1, NPOINT), jnp.int32))
    _, acc = lax.fori_loop(0, NPOINT, body, init)
    o_ref[0] = acc


def _fps(xyz, npoint):
    px = xyz[:, :, 0].reshape(B, _FR, _FC)
    py = xyz[:, :, 1].reshape(B, _FR, _FC)
    pz = xyz[:, :, 2].reshape(B, _FR, _FC)
    spec = pl.BlockSpec((1, _FR, _FC), lambda b: (b, 0, 0))
    out = pl.pallas_call(
        _fps_body,
        grid=(B,),
        in_specs=[spec, spec, spec],
        out_specs=pl.BlockSpec((1, 1, npoint), lambda b: (b, 0, 0)),
        out_shape=jax.ShapeDtypeStruct((B, 1, npoint), jnp.int32),
        scratch_shapes=[pltpu.VMEM((1, _FR, _FC), jnp.float32)],
    )(px, py, pz)
    return out[:, 0, :]


def _group(idx, xyz, new_xyz, features):
    grouped_xyz = jax.vmap(lambda pts, ix: pts[ix])(xyz, idx)
    grouped_xyz = grouped_xyz - new_xyz[:, :, None, :]
    grouped_xyz = jnp.transpose(grouped_xyz, (0, 3, 1, 2))
    grouped_feat = jax.vmap(lambda f, ix: f[:, ix])(features, idx)
    return jnp.concatenate([grouped_xyz, grouped_feat], axis=1)


def _mlp_branch(x, layers):
    for L in layers:
        x = jnp.einsum('oc,bcms->boms', L["W"], x)
        mean = jnp.mean(x, axis=(0, 2, 3), keepdims=True)
        var = jnp.var(x, axis=(0, 2, 3), keepdims=True)
        x = (x - mean) / jnp.sqrt(var + EPS)
        x = x * L["g"][None, :, None, None] + L["b"][None, :, None, None]
        x = jax.nn.relu(x)
    return x


def _out_kernel(x_ref, w_ref, g_ref, b_ref, o_ref):
    ys = []
    for b in range(B):
        ys.append(jnp.dot(w_ref[...], x_ref[b], preferred_element_type=jnp.float32))
    y = jnp.stack(ys, axis=0)
    cnt = y.shape[0] * y.shape[2]
    mean = jnp.sum(y, axis=(0, 2)) / cnt
    var = jnp.sum(y * y, axis=(0, 2)) / cnt - mean * mean
    scale = g_ref[...] / jnp.sqrt(var + EPS)
    shift = b_ref[...] - mean * scale
    o_ref[...] = jnp.maximum(y * scale[None, :, None] + shift[None, :, None], 0.0)


def _out_layer(nf, L):
    O, Cin = L["W"].shape
    M = nf.shape[2]
    return pl.pallas_call(
        _out_kernel,
        out_shape=jax.ShapeDtypeStruct((B, O, M), jnp.float32),
    )(nf, L["W"], L["g"], L["b"])


def kernel(xyz, features, params):
    fps_idx = _fps(xyz, NPOINT)
    new_xyz = jax.vmap(lambda pts, ix: pts[ix])(xyz, fps_idx)
    d2 = _pairwise_d2(xyz, new_xyz)
    feats = []
    for i in range(len(RADII)):
        c = jnp.cumsum((d2 <= RADII[i] * RADII[i]).astype(jnp.int32), axis=-1)
        q = jnp.arange(1, NSAMPLES[i] + 1, dtype=jnp.int32)
        idx_sorted = jax.vmap(jax.vmap(
            lambda cr: jnp.searchsorted(cr, q, side='left')))(c).astype(jnp.int32)
        first = idx_sorted[..., :1]
        idx = jnp.where(idx_sorted >= N, jnp.broadcast_to(first, idx_sorted.shape), idx_sorted)
        idx = jnp.where(idx >= N, 0, idx).astype(jnp.int32)
        g = _group(idx, xyz, new_xyz, features)
        h = _mlp_branch(g, params["branches"][i])
        h = jnp.max(h, axis=-1)
        feats.append(h)
    nf = jnp.concatenate(feats, axis=1)
    nf = _out_layer(nf, params["out"])
    return (new_xyz, nf)


# speedup vs baseline: 6.9895x; 1.4012x over previous
import functools
import jax
import jax.numpy as jnp
import numpy as np
from jax import lax
from jax.experimental import pallas as pl
from jax.experimental.pallas import tpu as pltpu

B, N, C = 4, 16384, 64
NPOINT = 1024
RADII = [0.5, 1.0, 2.0]
NSAMPLES = [16, 16, 32]
EPS = 1e-5

_CT = 64


def _d2_body(px_ref, py_ref, pz_ref, cx_ref, cy_ref, cz_ref,
             o0_ref, o1_ref, o2_ref):
    dx = cx_ref[...] - px_ref[...]
    dy = cy_ref[...] - py_ref[...]
    dz = cz_ref[...] - pz_ref[...]
    d2 = dx * dx + dy * dy + dz * dz
    for r, o_ref in ((RADII[0], o0_ref), (RADII[1], o1_ref), (RADII[2], o2_ref)):
        x = (d2 <= r * r).astype(jnp.int32)
        s = 1
        while s < N:
            pad = jnp.zeros(x.shape[:-1] + (s,), x.dtype)
            x = x + jnp.concatenate([pad, x[..., :-s]], axis=-1)
            s *= 2
        o_ref[...] = x


def _ball_counts(xyz, new_xyz):
    px = xyz[:, :, 0][:, None, :]
    py = xyz[:, :, 1][:, None, :]
    pz = xyz[:, :, 2][:, None, :]
    cx = new_xyz[:, :, 0][:, :, None]
    cy = new_xyz[:, :, 1][:, :, None]
    cz = new_xyz[:, :, 2][:, :, None]
    pt_spec = pl.BlockSpec((1, 1, N), lambda b, m: (b, 0, 0))
    c_spec = pl.BlockSpec((1, _CT, 1), lambda b, m: (b, m, 0))
    o_spec = pl.BlockSpec((1, _CT, N), lambda b, m: (b, m, 0))
    o_shape = jax.ShapeDtypeStruct((B, NPOINT, N), jnp.int32)
    return pl.pallas_call(
        _d2_body,
        grid=(B, NPOINT // _CT),
        in_specs=[pt_spec, pt_spec, pt_spec, c_spec, c_spec, c_spec],
        out_specs=[o_spec, o_spec, o_spec],
        out_shape=[o_shape, o_shape, o_shape],
    )(px, py, pz, cx, cy, cz)


_FR, _FC = 8, N // 8


def _fps_body(px_ref, py_ref, pz_ref, o_ref, dist_ref):
    nmat = (lax.broadcasted_iota(jnp.int32, (1, _FR, _FC), 1) * _FC
            + lax.broadcasted_iota(jnp.int32, (1, _FR, _FC), 2))
    px = px_ref[...]
    py = py_ref[...]
    pz = pz_ref[...]
    dist_ref[...] = jnp.full((1, _FR, _FC), 1e10, jnp.float32)

    iota_np = lax.broadcasted_iota(jnp.int32, (1, NPOINT), 1)

    def body(i, st):
        farthest, acc = st
        acc = jnp.where(iota_np == i, farthest, acc)
        sel = nmat == farthest
        zf = jnp.zeros((1, _FR, _FC), jnp.float32)
        cx = jnp.sum(jnp.where(sel, px, zf))
        cy = jnp.sum(jnp.where(sel, py, zf))
        cz = jnp.sum(jnp.where(sel, pz, zf))
        dx = px - cx
        dy = py - cy
        dz = pz - cz
        d = jnp.minimum(dist_ref[...], dx * dx + dy * dy + dz * dz)
        dist_ref[...] = d
        m = jnp.max(d)
        nxt = jnp.min(jnp.where(d == m, nmat, N)).astype(jnp.int32)
        return (nxt, acc)

    init = (jnp.int32(0), jnp.zeros((1, NPOINT), jnp.int32))
    _, acc = lax.fori_loop(0, NPOINT, body, init)
    o_ref[0] = acc


def _fps(xyz, npoint):
    px = xyz[:, :, 0].reshape(B, _FR, _FC)
    py = xyz[:, :, 1].reshape(B, _FR, _FC)
    pz = xyz[:, :, 2].reshape(B, _FR, _FC)
    spec = pl.BlockSpec((1, _FR, _FC), lambda b: (b, 0, 0))
    out = pl.pallas_call(
        _fps_body,
        grid=(B,),
        in_specs=[spec, spec, spec],
        out_specs=pl.BlockSpec((1, 1, npoint), lambda b: (b, 0, 0)),
        out_shape=jax.ShapeDtypeStruct((B, 1, npoint), jnp.int32),
        scratch_shapes=[pltpu.VMEM((1, _FR, _FC), jnp.float32)],
    )(px, py, pz)
    return out[:, 0, :]


def _group(idx, xyz, new_xyz, features):
    grouped_xyz = jax.vmap(lambda pts, ix: pts[ix])(xyz, idx)
    grouped_xyz = grouped_xyz - new_xyz[:, :, None, :]
    grouped_xyz = jnp.transpose(grouped_xyz, (0, 3, 1, 2))
    grouped_feat = jax.vmap(lambda f, ix: f[:, ix])(features, idx)
    return jnp.concatenate([grouped_xyz, grouped_feat], axis=1)


def _mlp_branch(x, layers):
    for L in layers:
        x = jnp.einsum('oc,bcms->boms', L["W"], x)
        mean = jnp.mean(x, axis=(0, 2, 3), keepdims=True)
        var = jnp.var(x, axis=(0, 2, 3), keepdims=True)
        x = (x - mean) / jnp.sqrt(var + EPS)
        x = x * L["g"][None, :, None, None] + L["b"][None, :, None, None]
        x = jax.nn.relu(x)
    return x


def _out_kernel(x_ref, w_ref, g_ref, b_ref, o_ref):
    ys = []
    for b in range(B):
        ys.append(jnp.dot(w_ref[...], x_ref[b], preferred_element_type=jnp.float32))
    y = jnp.stack(ys, axis=0)
    cnt = y.shape[0] * y.shape[2]
    mean = jnp.sum(y, axis=(0, 2)) / cnt
    var = jnp.sum(y * y, axis=(0, 2)) / cnt - mean * mean
    scale = g_ref[...] / jnp.sqrt(var + EPS)
    shift = b_ref[...] - mean * scale
    o_ref[...] = jnp.maximum(y * scale[None, :, None] + shift[None, :, None], 0.0)


def _out_layer(nf, L):
    O, Cin = L["W"].shape
    M = nf.shape[2]
    return pl.pallas_call(
        _out_kernel,
        out_shape=jax.ShapeDtypeStruct((B, O, M), jnp.float32),
    )(nf, L["W"], L["g"], L["b"])


def kernel(xyz, features, params):
    fps_idx = _fps(xyz, NPOINT)
    new_xyz = jax.vmap(lambda pts, ix: pts[ix])(xyz, fps_idx)
    counts = _ball_counts(xyz, new_xyz)
    feats = []
    for i in range(len(RADII)):
        c = counts[i]
        q = jnp.arange(1, NSAMPLES[i] + 1, dtype=jnp.int32)
        idx_sorted = jax.vmap(jax.vmap(
            lambda cr: jnp.searchsorted(cr, q, side='left')))(c).astype(jnp.int32)
        first = idx_sorted[..., :1]
        idx = jnp.where(idx_sorted >= N, jnp.broadcast_to(first, idx_sorted.shape), idx_sorted)
        idx = jnp.where(idx >= N, 0, idx).astype(jnp.int32)
        g = _group(idx, xyz, new_xyz, features)
        h = _mlp_branch(g, params["branches"][i])
        h = jnp.max(h, axis=-1)
        feats.append(h)
    nf = jnp.concatenate(feats, axis=1)
    nf = _out_layer(nf, params["out"])
    return (new_xyz, nf)
